# baseline (speedup 1.0000x reference)
import jax
import jax.numpy as jnp
from jax.experimental import pallas as pl

_B, _S, _D, _E, _K = 4, 4096, 2048, 16, 2
_M = _B * _S
_BM = 512


def _router_body(x_ref, wt_ref, w_out_ref, i_out_ref, logits_ref):
    logits = jnp.dot(x_ref[...], wt_ref[...], preferred_element_type=jnp.float32)
    logits_ref[...] = logits

    m1 = jnp.max(logits, axis=-1)
    i1 = jnp.argmax(logits, axis=-1).astype(jnp.int32)
    lane = jax.lax.broadcasted_iota(jnp.int32, logits.shape, 1)
    masked = jnp.where(lane == i1[:, None], -jnp.inf, logits)
    m2 = jnp.max(masked, axis=-1)
    i2 = jnp.argmax(masked, axis=-1).astype(jnp.int32)

    e2 = jnp.exp(m2 - m1)
    denom = 1.0 + e2
    w1 = 1.0 / denom
    w2 = e2 / denom

    w_out_ref[...] = jnp.stack([w1, w2], axis=-1)
    i_out_ref[...] = jnp.stack([i1, i2], axis=-1)


@jax.jit
def kernel(x, W):
    xm = x.reshape(_M, _D)
    wt = W.T

    grid = (_M // _BM,)
    w_out, i_out, logits = pl.pallas_call(
        _router_body,
        grid=grid,
        in_specs=[
            pl.BlockSpec((_BM, _D), lambda i: (i, 0)),
            pl.BlockSpec((_D, _E), lambda i: (0, 0)),
        ],
        out_specs=[
            pl.BlockSpec((_BM, _K), lambda i: (i, 0)),
            pl.BlockSpec((_BM, _K), lambda i: (i, 0)),
            pl.BlockSpec((_BM, _E), lambda i: (i, 0)),
        ],
        out_shape=[
            jax.ShapeDtypeStruct((_M, _K), jnp.float32),
            jax.ShapeDtypeStruct((_M, _K), jnp.int32),
            jax.ShapeDtypeStruct((_M, _E), jnp.float32),
        ],
    )(xm, wt)

    return (
        w_out.reshape(_B, _S, _K),
        i_out.reshape(_B, _S, _K),
        logits.reshape(_B, _S, _E),
    )


# speedup vs baseline: 1.1314x; 1.1314x over previous
import jax
import jax.numpy as jnp
from jax.experimental import pallas as pl

_B, _S, _D, _E, _K = 4, 4096, 2048, 16, 2
_M = _B * _S
_BM = 2048


def _router_body(x_ref, wt_ref, w_out_ref, i_out_ref, logits_ref):
    logits = jnp.dot(x_ref[...], wt_ref[...], preferred_element_type=jnp.float32)
    logits_ref[...] = logits

    m1 = jnp.max(logits, axis=-1)
    i1 = jnp.argmax(logits, axis=-1).astype(jnp.int32)
    lane = jax.lax.broadcasted_iota(jnp.int32, logits.shape, 1)
    masked = jnp.where(lane == i1[:, None], -jnp.inf, logits)
    m2 = jnp.max(masked, axis=-1)
    i2 = jnp.argmax(masked, axis=-1).astype(jnp.int32)

    e2 = jnp.exp(m2 - m1)
    denom = 1.0 + e2
    w1 = 1.0 / denom
    w2 = e2 / denom

    w_out_ref[...] = jnp.stack([w1, w2], axis=-1)
    i_out_ref[...] = jnp.stack([i1, i2], axis=-1)


@jax.jit
def kernel(x, W):
    xm = x.reshape(_M, _D)
    wt = W.T

    grid = (_M // _BM,)
    w_out, i_out, logits = pl.pallas_call(
        _router_body,
        grid=grid,
        in_specs=[
            pl.BlockSpec((_BM, _D), lambda i: (i, 0)),
            pl.BlockSpec((_D, _E), lambda i: (0, 0)),
        ],
        out_specs=[
            pl.BlockSpec((_BM, _K), lambda i: (i, 0)),
            pl.BlockSpec((_BM, _K), lambda i: (i, 0)),
            pl.BlockSpec((_BM, _E), lambda i: (i, 0)),
        ],
        out_shape=[
            jax.ShapeDtypeStruct((_M, _K), jnp.float32),
            jax.ShapeDtypeStruct((_M, _K), jnp.int32),
            jax.ShapeDtypeStruct((_M, _E), jnp.float32),
        ],
    )(xm, wt)

    return (
        w_out.reshape(_B, _S, _K),
        i_out.reshape(_B, _S, _K),
        logits.reshape(_B, _S, _E),
    )
